# baseline (speedup 1.0000x reference)
import functools

import jax
import jax.numpy as jnp
from jax import lax
from jax.experimental import pallas as pl
from jax.experimental.pallas import tpu as pltpu
from jax.experimental.pallas import tpu_sc as plsc

BLOCK = 256
N_WORKERS = 32
GATHER_CHUNK = 32
COMBINE_CHUNK = 16


def _routing_metadata(top_k_index, top_k_weights, E, B):
    T, K = top_k_index.shape
    S = T * K
    S_pad = S + E * B
    NB = S_pad // B
    flat_e = top_k_index.reshape(S)
    oh = (flat_e[:, None] == jnp.arange(E, dtype=jnp.int32)[None, :]).astype(jnp.int32)
    counts = oh.sum(axis=0)
    rank = jnp.sum(jnp.cumsum(oh, axis=0) * oh, axis=1) - 1
    padded_counts = ((counts + B - 1) // B) * B
    pad_end = jnp.cumsum(padded_counts)
    pad_off = pad_end - padded_counts
    pos = (jnp.take(pad_off, flat_e) + rank).astype(jnp.int32)
    total_pad = pad_end[-1]
    blk_starts = jnp.arange(NB, dtype=jnp.int32) * B
    last_active = jnp.sum((pad_end <= total_pad - 1).astype(jnp.int32))
    block_expert = jnp.minimum(
        jnp.sum((pad_end[None, :] <= blk_starts[:, None]).astype(jnp.int32), axis=1),
        last_active).astype(jnp.int32)
    num_active = (total_pad // B).astype(jnp.int32).reshape(1)
    g = pos.reshape(T, K)
    return pos, block_expert, num_active, g[:, 0], g[:, 1], S_pad, NB


def _meta_body(tki_ref, g0_ref, g1_ref, be_ref, na_ref, *, E, B, NB, T, G, J):
    a = tki_ref[0]
    b = tki_ref[1]
    e_iota = lax.broadcasted_iota(jnp.int32, (E, G, J), 0)
    oh0 = (a[None] == e_iota).astype(jnp.float32)
    oh1 = (b[None] == e_iota).astype(jnp.float32)

    row = lax.broadcasted_iota(jnp.int32, (J, J), 0)
    col = lax.broadcasted_iota(jnp.int32, (J, J), 1)
    U_incl = (row <= col).astype(jnp.float32)
    rowg = lax.broadcasted_iota(jnp.int32, (G, G), 0)
    colg = lax.broadcasted_iota(jnp.int32, (G, G), 1)
    Ug_strict = (rowg < colg).astype(jnp.float32)
    Lg_incl = (colg <= rowg).astype(jnp.float32)

    def excl_cumsum(oh):
        incl = lax.dot_general(oh, U_incl, (((2,), (0,)), ((), ())),
                               preferred_element_type=jnp.float32)
        totg = incl[:, :, J - 1]
        goff = lax.dot_general(totg, Ug_strict, (((1,), (0,)), ((), ())),
                               preferred_element_type=jnp.float32)
        return incl - oh + goff[:, :, None], totg

    ec0, totg0 = excl_cumsum(oh0)
    ec1, totg1 = excl_cumsum(oh1)
    ec = ec0 + ec1

    counts = jnp.sum(totg0 + totg1, axis=1, keepdims=True)
    padded = jnp.floor((counts + (B - 1)) * (1.0 / B)).astype(jnp.float32) * B
    pad_end = lax.dot_general(Lg_incl[:E, :E], padded, (((1,), (0,)), ((), ())),
                              preferred_element_type=jnp.float32)
    pad_off = pad_end - padded

    g0 = jnp.sum(oh0 * (ec + pad_off[:, :, None]), axis=0)
    g1 = (jnp.sum(oh1 * (ec + pad_off[:, :, None]), axis=0)
          + (a == b).astype(jnp.float32))
    g0_ref[...] = g0.astype(jnp.int32)
    g1_ref[...] = g1.astype(jnp.int32)

    total = jnp.sum(padded)
    blk_starts = (lax.broadcasted_iota(jnp.int32, (E, NB), 1) * B
                  ).astype(jnp.float32)
    be = jnp.sum((pad_end <= blk_starts).astype(jnp.float32), axis=0,
                 keepdims=True)
    last_active = jnp.sum((pad_end <= total - 1.0).astype(jnp.float32))
    be_ref[...] = jnp.minimum(be, last_active).astype(jnp.int32)
    na_ref[...] = jnp.reshape((total * (1.0 / B)).astype(jnp.int32), (1, 1))


def _meta_kernel(top_k_index, E, B, NB, interpret=False):
    T, K = top_k_index.shape
    assert K == 2
    G, J = 16, 128
    assert G * J == T
    tki3 = top_k_index.T.reshape(K, G, J)
    body = functools.partial(_meta_body, E=E, B=B, NB=NB, T=T, G=G, J=J)
    g0, g1, be, na = pl.pallas_call(
        body,
        out_shape=(jax.ShapeDtypeStruct((G, J), jnp.int32),
                   jax.ShapeDtypeStruct((G, J), jnp.int32),
                   jax.ShapeDtypeStruct((1, NB), jnp.int32),
                   jax.ShapeDtypeStruct((1, 1), jnp.int32)),
        interpret=interpret,
    )(tki3)
    return (g0.reshape(T), g1.reshape(T), be.reshape(NB), na.reshape(1))


def _mlp_body(be_ref, na_ref, x_ref, gu_ref, dn_ref, y_ref):
    inter = dn_ref.shape[2]
    b = pl.program_id(0)

    @pl.when(b < na_ref[0])
    def _():
        x = x_ref[...]
        gu = gu_ref[0]
        h = lax.dot_general(x, gu, (((1,), (1,)), ((), ())),
                            preferred_element_type=jnp.float32)
        gate = h[:, :inter]
        up = h[:, inter:]
        act = gate * jax.nn.sigmoid(gate) * up
        dn = dn_ref[0]
        y_ref[...] = lax.dot_general(act, dn, (((1,), (1,)), ((), ())),
                                     preferred_element_type=jnp.float32)


def _grouped_mlp(x_pad, gate_up_proj, down_proj, block_expert, num_active,
                 S_pad, NB, B):
    E, twoI, H = gate_up_proj.shape
    I = twoI // 2
    grid_spec = pltpu.PrefetchScalarGridSpec(
        num_scalar_prefetch=2,
        grid=(NB,),
        in_specs=[
            pl.BlockSpec((B, H), lambda b, be, na: (b, 0)),
            pl.BlockSpec((1, twoI, H), lambda b, be, na: (be[b], 0, 0)),
            pl.BlockSpec((1, H, I), lambda b, be, na: (be[b], 0, 0)),
        ],
        out_specs=pl.BlockSpec((B, H), lambda b, be, na: (b, 0)),
    )
    return pl.pallas_call(
        _mlp_body,
        grid_spec=grid_spec,
        out_shape=jax.ShapeDtypeStruct((S_pad, H), jnp.float32),
        compiler_params=pltpu.CompilerParams(
            dimension_semantics=("arbitrary",)),
    )(block_expert, num_active, x_pad, gate_up_proj, down_proj)


def _make_dispatch(T, S_pad, H):
    toks_pw = T // N_WORKERS
    chunk = 16
    n_chunks = toks_pw // chunk
    mesh = plsc.VectorSubcoreMesh(core_axis_name="c", subcore_axis_name="s")

    @functools.partial(
        pl.kernel, mesh=mesh,
        out_type=jax.ShapeDtypeStruct((S_pad, H), jnp.float32),
        scratch_types=[
            pltpu.VMEM((n_chunks, chunk), jnp.int32),
            pltpu.VMEM((n_chunks, chunk), jnp.int32),
            pltpu.VMEM((2, chunk, H), jnp.float32),
            pltpu.SemaphoreType.DMA,
            pltpu.SemaphoreType.DMA,
        ],
    )
    def dispatch_k(g0_hbm, g1_hbm, hid_hbm, out_hbm, i0_v, i1_v, rows_v,
                   lsem, ssem):
        wid = lax.axis_index("s") * 2 + lax.axis_index("c")
        base = wid * toks_pw
        for c in range(n_chunks):
            pltpu.sync_copy(g0_hbm.at[pl.ds(base + c * chunk, chunk)],
                            i0_v.at[c])
            pltpu.sync_copy(g1_hbm.at[pl.ds(base + c * chunk, chunk)],
                            i1_v.at[c])

        def load(i):
            pltpu.async_copy(hid_hbm.at[pl.ds(base + i * chunk, chunk)],
                             rows_v.at[i % 2], lsem)

        def wait_scatters(i):
            pltpu.make_async_copy(rows_v.at[i % 2], out_hbm.at[i0_v.at[i]],
                                  ssem).wait()
            pltpu.make_async_copy(rows_v.at[i % 2], out_hbm.at[i1_v.at[i]],
                                  ssem).wait()

        load(0)
        for i in range(n_chunks):
            pltpu.make_async_copy(hid_hbm.at[pl.ds(base + i * chunk, chunk)],
                                  rows_v.at[i % 2], lsem).wait()
            if i >= 1:
                wait_scatters(i - 1)
            if i + 1 < n_chunks:
                load(i + 1)
            pltpu.async_copy(rows_v.at[i % 2], out_hbm.at[i0_v.at[i]], ssem)
            pltpu.async_copy(rows_v.at[i % 2], out_hbm.at[i1_v.at[i]], ssem)
        wait_scatters(n_chunks - 1)

    return dispatch_k


def _make_combine(T, H, S_pad):
    toks_pw = T // N_WORKERS
    chunk = 8
    n_chunks = toks_pw // chunk
    L = 16
    mesh = plsc.VectorSubcoreMesh(core_axis_name="c", subcore_axis_name="s")

    @functools.partial(
        pl.kernel, mesh=mesh,
        out_type=jax.ShapeDtypeStruct((T, H), jnp.float32),
        scratch_types=[
            pltpu.VMEM((toks_pw,), jnp.int32),
            pltpu.VMEM((toks_pw,), jnp.int32),
            pltpu.VMEM((toks_pw,), jnp.float32),
            pltpu.VMEM((toks_pw,), jnp.float32),
            pltpu.VMEM((2, chunk, H), jnp.float32),
            pltpu.VMEM((2, chunk, H), jnp.float32),
            pltpu.VMEM((2, chunk, H), jnp.float32),
            pltpu.SemaphoreType.DMA,
            pltpu.SemaphoreType.DMA,
            pltpu.SemaphoreType.DMA,
            pltpu.SemaphoreType.DMA,
        ],
    )
    def combine_k(g0_hbm, g1_hbm, w0_hbm, w1_hbm, ypad_hbm, out_hbm,
                  i0_v, i1_v, w0_v, w1_v, a_v, b_v, o_v,
                  gsem0, gsem1, wsem0, wsem1):
        wid = lax.axis_index("s") * 2 + lax.axis_index("c")
        base = wid * toks_pw
        gsems = (gsem0, gsem1)
        wsems = (wsem0, wsem1)

        pltpu.sync_copy(g0_hbm.at[pl.ds(base, toks_pw)], i0_v)
        pltpu.sync_copy(g1_hbm.at[pl.ds(base, toks_pw)], i1_v)
        pltpu.sync_copy(w0_hbm.at[pl.ds(base, toks_pw)], w0_v)
        pltpu.sync_copy(w1_hbm.at[pl.ds(base, toks_pw)], w1_v)

        def fire(c):
            p = c % 2
            pltpu.async_copy(ypad_hbm.at[i0_v.at[pl.ds(c * chunk, chunk)]],
                             a_v.at[p], gsems[p])
            pltpu.async_copy(ypad_hbm.at[i1_v.at[pl.ds(c * chunk, chunk)]],
                             b_v.at[p], gsems[p])

        fire(0)
        for c in range(n_chunks):
            p = c % 2
            if c + 1 < n_chunks:
                fire(c + 1)
            pltpu.make_async_copy(ypad_hbm.at[i0_v.at[pl.ds(c * chunk, chunk)]],
                                  a_v.at[p], gsems[p]).wait()
            pltpu.make_async_copy(ypad_hbm.at[i1_v.at[pl.ds(c * chunk, chunk)]],
                                  b_v.at[p], gsems[p]).wait()
            if c >= 2:
                pltpu.make_async_copy(
                    o_v.at[p], out_hbm.at[pl.ds(base + (c - 2) * chunk, chunk)],
                    wsems[p]).wait()
            wv0 = w0_v[pl.ds((c // 2) * 16, 16)]
            wv1 = w1_v[pl.ds((c // 2) * 16, 16)]
            for r in range(chunk):
                w0s = wv0[(c % 2) * chunk + r]
                w1s = wv1[(c % 2) * chunk + r]

                @plsc.parallel_loop(0, H // L, unroll=8)
                def col(j):
                    o_v[p, r, pl.ds(j * L, L)] = (
                        a_v[p, r, pl.ds(j * L, L)] * w0s
                        + b_v[p, r, pl.ds(j * L, L)] * w1s)
            pltpu.async_copy(o_v.at[p],
                             out_hbm.at[pl.ds(base + c * chunk, chunk)],
                             wsems[p])
        for c in (n_chunks - 2, n_chunks - 1):
            p = c % 2
            pltpu.make_async_copy(
                o_v.at[p], out_hbm.at[pl.ds(base + c * chunk, chunk)],
                wsems[p]).wait()

    return combine_k


def kernel(hidden_states, top_k_index, top_k_weights, gate_up_proj, down_proj):
    T, H = hidden_states.shape
    K = top_k_index.shape[1]
    E = gate_up_proj.shape[0]
    B = BLOCK
    S = T * K
    S_pad = S + E * B
    NB = S_pad // B
    g0, g1, block_expert, num_active = _meta_kernel(top_k_index, E, B, NB)
    x_pad = _make_dispatch(T, S_pad, H)(g0, g1, hidden_states)
    y_pad = _grouped_mlp(x_pad, gate_up_proj, down_proj,
                         block_expert, num_active, S_pad, NB, B)
    w0 = top_k_weights[:, 0] + 0.0
    w1 = top_k_weights[:, 1] + 0.0
    return _make_combine(T, H, S_pad)(g0, g1, w0, w1, y_pad)


# speedup vs baseline: 1.0112x; 1.0112x over previous
import functools

import jax
import jax.numpy as jnp
from jax import lax
from jax.experimental import pallas as pl
from jax.experimental.pallas import tpu as pltpu
from jax.experimental.pallas import tpu_sc as plsc

BLOCK = 256
N_WORKERS = 32
GATHER_CHUNK = 32
COMBINE_CHUNK = 16


def _routing_metadata(top_k_index, top_k_weights, E, B):
    T, K = top_k_index.shape
    S = T * K
    S_pad = S + E * B
    NB = S_pad // B
    flat_e = top_k_index.reshape(S)
    oh = (flat_e[:, None] == jnp.arange(E, dtype=jnp.int32)[None, :]).astype(jnp.int32)
    counts = oh.sum(axis=0)
    rank = jnp.sum(jnp.cumsum(oh, axis=0) * oh, axis=1) - 1
    padded_counts = ((counts + B - 1) // B) * B
    pad_end = jnp.cumsum(padded_counts)
    pad_off = pad_end - padded_counts
    pos = (jnp.take(pad_off, flat_e) + rank).astype(jnp.int32)
    total_pad = pad_end[-1]
    blk_starts = jnp.arange(NB, dtype=jnp.int32) * B
    last_active = jnp.sum((pad_end <= total_pad - 1).astype(jnp.int32))
    block_expert = jnp.minimum(
        jnp.sum((pad_end[None, :] <= blk_starts[:, None]).astype(jnp.int32), axis=1),
        last_active).astype(jnp.int32)
    num_active = (total_pad // B).astype(jnp.int32).reshape(1)
    g = pos.reshape(T, K)
    return pos, block_expert, num_active, g[:, 0], g[:, 1], S_pad, NB


def _meta_body(tki_ref, g0_ref, g1_ref, be_ref, na_ref, *, E, B, NB, T, G, J):
    a = tki_ref[0]
    b = tki_ref[1]
    e_iota = lax.broadcasted_iota(jnp.int32, (E, G, J), 0)
    oh0 = (a[None] == e_iota).astype(jnp.float32)
    oh1 = (b[None] == e_iota).astype(jnp.float32)

    row = lax.broadcasted_iota(jnp.int32, (J, J), 0)
    col = lax.broadcasted_iota(jnp.int32, (J, J), 1)
    U_incl = (row <= col).astype(jnp.float32)
    rowg = lax.broadcasted_iota(jnp.int32, (G, G), 0)
    colg = lax.broadcasted_iota(jnp.int32, (G, G), 1)
    Ug_strict = (rowg < colg).astype(jnp.float32)
    Lg_incl = (colg <= rowg).astype(jnp.float32)

    def excl_cumsum(oh):
        incl = lax.dot_general(oh, U_incl, (((2,), (0,)), ((), ())),
                               preferred_element_type=jnp.float32)
        totg = incl[:, :, J - 1]
        goff = lax.dot_general(totg, Ug_strict, (((1,), (0,)), ((), ())),
                               preferred_element_type=jnp.float32)
        return incl - oh + goff[:, :, None], totg

    ec0, totg0 = excl_cumsum(oh0)
    ec1, totg1 = excl_cumsum(oh1)
    ec = ec0 + ec1

    counts = jnp.sum(totg0 + totg1, axis=1, keepdims=True)
    padded = jnp.floor((counts + (B - 1)) * (1.0 / B)).astype(jnp.float32) * B
    pad_end = lax.dot_general(Lg_incl[:E, :E], padded, (((1,), (0,)), ((), ())),
                              preferred_element_type=jnp.float32)
    pad_off = pad_end - padded

    g0 = jnp.sum(oh0 * (ec + pad_off[:, :, None]), axis=0)
    g1 = (jnp.sum(oh1 * (ec + pad_off[:, :, None]), axis=0)
          + (a == b).astype(jnp.float32))
    g0_ref[...] = g0.astype(jnp.int32)
    g1_ref[...] = g1.astype(jnp.int32)

    total = jnp.sum(padded)
    blk_starts = (lax.broadcasted_iota(jnp.int32, (E, NB), 1) * B
                  ).astype(jnp.float32)
    be = jnp.sum((pad_end <= blk_starts).astype(jnp.float32), axis=0,
                 keepdims=True)
    last_active = jnp.sum((pad_end <= total - 1.0).astype(jnp.float32))
    be_ref[...] = jnp.minimum(be, last_active).astype(jnp.int32)
    na_ref[...] = jnp.reshape((total * (1.0 / B)).astype(jnp.int32), (1, 1))


def _meta_kernel(top_k_index, E, B, NB, interpret=False):
    T, K = top_k_index.shape
    assert K == 2
    G, J = 16, 128
    assert G * J == T
    tki3 = top_k_index.T.reshape(K, G, J)
    body = functools.partial(_meta_body, E=E, B=B, NB=NB, T=T, G=G, J=J)
    g0, g1, be, na = pl.pallas_call(
        body,
        out_shape=(jax.ShapeDtypeStruct((G, J), jnp.int32),
                   jax.ShapeDtypeStruct((G, J), jnp.int32),
                   jax.ShapeDtypeStruct((1, NB), jnp.int32),
                   jax.ShapeDtypeStruct((1, 1), jnp.int32)),
        interpret=interpret,
    )(tki3)
    return (g0.reshape(T), g1.reshape(T), be.reshape(NB), na.reshape(1))


def _mlp_body(be_ref, na_ref, x_ref, gu_ref, dn_ref, y_ref):
    inter = dn_ref.shape[2]
    b = pl.program_id(0)

    @pl.when(b < na_ref[0])
    def _():
        x = x_ref[...]
        gu = gu_ref[0]
        h = lax.dot_general(x, gu, (((1,), (1,)), ((), ())),
                            preferred_element_type=jnp.float32)
        gate = h[:, :inter]
        up = h[:, inter:]
        act = gate * jax.nn.sigmoid(gate) * up
        dn = dn_ref[0]
        y_ref[...] = lax.dot_general(act, dn, (((1,), (1,)), ((), ())),
                                     preferred_element_type=jnp.float32)


def _grouped_mlp(x_pad, gate_up_proj, down_proj, block_expert, num_active,
                 S_pad, NB, B):
    E, twoI, H = gate_up_proj.shape
    I = twoI // 2
    grid_spec = pltpu.PrefetchScalarGridSpec(
        num_scalar_prefetch=2,
        grid=(NB,),
        in_specs=[
            pl.BlockSpec((B, H), lambda b, be, na: (b, 0)),
            pl.BlockSpec((1, twoI, H), lambda b, be, na: (be[b], 0, 0)),
            pl.BlockSpec((1, H, I), lambda b, be, na: (be[b], 0, 0)),
        ],
        out_specs=pl.BlockSpec((B, H), lambda b, be, na: (b, 0)),
    )
    return pl.pallas_call(
        _mlp_body,
        grid_spec=grid_spec,
        out_shape=jax.ShapeDtypeStruct((S_pad, H), jnp.float32),
        compiler_params=pltpu.CompilerParams(
            dimension_semantics=("arbitrary",)),
    )(block_expert, num_active, x_pad, gate_up_proj, down_proj)


def _make_dispatch(T, S_pad, H):
    toks_pw = T // N_WORKERS
    chunk = GATHER_CHUNK
    n_chunks = toks_pw // chunk
    mesh = plsc.VectorSubcoreMesh(core_axis_name="c", subcore_axis_name="s")

    @functools.partial(
        pl.kernel, mesh=mesh,
        out_type=jax.ShapeDtypeStruct((S_pad, H), jnp.float32),
        scratch_types=[
            pltpu.VMEM((chunk,), jnp.int32),
            pltpu.VMEM((chunk,), jnp.int32),
            pltpu.VMEM((chunk, H), jnp.float32),
            pltpu.SemaphoreType.DMA,
        ],
    )
    def dispatch_k(g0_hbm, g1_hbm, hid_hbm, out_hbm, i0_v, i1_v, rows_v, sem):
        wid = lax.axis_index("s") * 2 + lax.axis_index("c")
        base = wid * toks_pw

        def body(i, carry):
            off = base + i * chunk
            pltpu.sync_copy(g0_hbm.at[pl.ds(off, chunk)], i0_v)
            pltpu.sync_copy(g1_hbm.at[pl.ds(off, chunk)], i1_v)
            pltpu.sync_copy(hid_hbm.at[pl.ds(off, chunk)], rows_v)
            cp0 = pltpu.async_copy(rows_v, out_hbm.at[i0_v], sem)
            cp1 = pltpu.async_copy(rows_v, out_hbm.at[i1_v], sem)
            cp0.wait()
            cp1.wait()
            return carry

        lax.fori_loop(0, n_chunks, body, 0)

    return dispatch_k


def _make_combine(T, H, S_pad):
    toks_pw = T // N_WORKERS
    chunk = 8
    n_chunks = toks_pw // chunk
    L = 16
    mesh = plsc.VectorSubcoreMesh(core_axis_name="c", subcore_axis_name="s")

    @functools.partial(
        pl.kernel, mesh=mesh,
        out_type=jax.ShapeDtypeStruct((T, H), jnp.float32),
        scratch_types=[
            pltpu.VMEM((toks_pw,), jnp.int32),
            pltpu.VMEM((toks_pw,), jnp.int32),
            pltpu.VMEM((toks_pw,), jnp.float32),
            pltpu.VMEM((toks_pw,), jnp.float32),
            pltpu.VMEM((2, chunk, H), jnp.float32),
            pltpu.VMEM((2, chunk, H), jnp.float32),
            pltpu.VMEM((2, chunk, H), jnp.float32),
            pltpu.SemaphoreType.DMA,
            pltpu.SemaphoreType.DMA,
            pltpu.SemaphoreType.DMA,
            pltpu.SemaphoreType.DMA,
        ],
    )
    def combine_k(g0_hbm, g1_hbm, w0_hbm, w1_hbm, ypad_hbm, out_hbm,
                  i0_v, i1_v, w0_v, w1_v, a_v, b_v, o_v,
                  gsem0, gsem1, wsem0, wsem1):
        wid = lax.axis_index("s") * 2 + lax.axis_index("c")
        base = wid * toks_pw
        gsems = (gsem0, gsem1)
        wsems = (wsem0, wsem1)

        pltpu.sync_copy(g0_hbm.at[pl.ds(base, toks_pw)], i0_v)
        pltpu.sync_copy(g1_hbm.at[pl.ds(base, toks_pw)], i1_v)
        pltpu.sync_copy(w0_hbm.at[pl.ds(base, toks_pw)], w0_v)
        pltpu.sync_copy(w1_hbm.at[pl.ds(base, toks_pw)], w1_v)

        def fire(c):
            p = c % 2
            pltpu.async_copy(ypad_hbm.at[i0_v.at[pl.ds(c * chunk, chunk)]],
                             a_v.at[p], gsems[p])
            pltpu.async_copy(ypad_hbm.at[i1_v.at[pl.ds(c * chunk, chunk)]],
                             b_v.at[p], gsems[p])

        fire(0)
        for c in range(n_chunks):
            p = c % 2
            if c + 1 < n_chunks:
                fire(c + 1)
            pltpu.make_async_copy(ypad_hbm.at[i0_v.at[pl.ds(c * chunk, chunk)]],
                                  a_v.at[p], gsems[p]).wait()
            pltpu.make_async_copy(ypad_hbm.at[i1_v.at[pl.ds(c * chunk, chunk)]],
                                  b_v.at[p], gsems[p]).wait()
            if c >= 2:
                pltpu.make_async_copy(
                    o_v.at[p], out_hbm.at[pl.ds(base + (c - 2) * chunk, chunk)],
                    wsems[p]).wait()
            wv0 = w0_v[pl.ds((c // 2) * 16, 16)]
            wv1 = w1_v[pl.ds((c // 2) * 16, 16)]
            for r in range(chunk):
                w0s = wv0[(c % 2) * chunk + r]
                w1s = wv1[(c % 2) * chunk + r]

                @plsc.parallel_loop(0, H // L, unroll=8)
                def col(j):
                    o_v[p, r, pl.ds(j * L, L)] = (
                        a_v[p, r, pl.ds(j * L, L)] * w0s
                        + b_v[p, r, pl.ds(j * L, L)] * w1s)
            pltpu.async_copy(o_v.at[p],
                             out_hbm.at[pl.ds(base + c * chunk, chunk)],
                             wsems[p])
        for c in (n_chunks - 2, n_chunks - 1):
            p = c % 2
            pltpu.make_async_copy(
                o_v.at[p], out_hbm.at[pl.ds(base + c * chunk, chunk)],
                wsems[p]).wait()

    return combine_k


def kernel(hidden_states, top_k_index, top_k_weights, gate_up_proj, down_proj):
    T, H = hidden_states.shape
    K = top_k_index.shape[1]
    E = gate_up_proj.shape[0]
    B = BLOCK
    S = T * K
    S_pad = S + E * B
    NB = S_pad // B
    g0, g1, block_expert, num_active = _meta_kernel(top_k_index, E, B, NB)
    x_pad = _make_dispatch(T, S_pad, H)(g0, g1, hidden_states)
    y_pad = _grouped_mlp(x_pad, gate_up_proj, down_proj,
                         block_expert, num_active, S_pad, NB, B)
    w0 = top_k_weights[:, 0] + 0.0
    w1 = top_k_weights[:, 1] + 0.0
    return _make_combine(T, H, S_pad)(g0, g1, w0, w1, y_pad)
